# Initial kernel scaffold; baseline (speedup 1.0000x reference)
import jax
import jax.numpy as jnp
from jax.experimental import pallas as pl


def kernel(x, table):
    raise NotImplementedError("write your pallas kernel here")


# speedup vs baseline: 8.5335x; 8.5335x over previous
import functools

import jax
import jax.numpy as jnp
from jax import lax
from jax.experimental import pallas as pl
from jax.experimental.pallas import tpu as pltpu
from jax.experimental.pallas import tpu_sc as plsc

NC = 2
NS = 16
NW = NC * NS

BATCH = 4096
HIST = 50
EMBED_DIM = 64
VOCAB = 100000

B_PER_W = BATCH // NW
CB = 2
CHUNK_IDX = CB * HIST
N_CHUNKS = B_PER_W // CB
NQ = EMBED_DIM // 16


def _accumulate(buf, out_v, c):
  for b in range(CB):
    rb = c * CB + b

    def body(l, accs):
      r = b * HIST + l
      return tuple(accs[q] + buf[r, pl.ds(16 * q, 16)] for q in range(NQ))

    z = jnp.zeros((16,), jnp.float32)
    accs = lax.fori_loop(0, HIST, body, (z,) * NQ)
    for q in range(NQ):
      out_v[rb, pl.ds(16 * q, 16)] = accs[q]


def _sc_body(x_hbm, table_hbm, out_hbm, idx_v, buf0, buf1, out_v, sem0, sem1):
  cid = lax.axis_index("c")
  sid = lax.axis_index("s")
  wid = sid * NC + cid

  pltpu.sync_copy(x_hbm.at[wid], idx_v)

  bufs = (buf0, buf1)
  sems = (sem0, sem1)

  pltpu.async_copy(table_hbm.at[idx_v.at[0]], buf0, sem0)
  pltpu.async_copy(table_hbm.at[idx_v.at[1]], buf1, sem1)

  def outer(c2, carry):
    for phase in range(2):
      c = 2 * c2 + phase
      buf, sem = bufs[phase], sems[phase]
      pltpu.make_async_copy(table_hbm.at[idx_v.at[0]], buf, sem).wait()
      _accumulate(buf, out_v, c)
      pltpu.async_copy(table_hbm.at[idx_v.at[c + 2]], buf, sem)
    return carry

  lax.fori_loop(0, N_CHUNKS // 2 - 1, outer, 0)
  for phase in range(2):
    c = N_CHUNKS - 2 + phase
    buf, sem = bufs[phase], sems[phase]
    pltpu.make_async_copy(table_hbm.at[idx_v.at[0]], buf, sem).wait()
    _accumulate(buf, out_v, c)

  pltpu.sync_copy(out_v, out_hbm.at[wid])


def kernel(x, table):
  x3 = x.reshape(NW, N_CHUNKS, CHUNK_IDX)
  mesh = plsc.VectorSubcoreMesh(core_axis_name="c", subcore_axis_name="s")
  run = functools.partial(
      pl.kernel,
      out_type=jax.ShapeDtypeStruct((NW, B_PER_W, EMBED_DIM), jnp.float32),
      mesh=mesh,
      compiler_params=pltpu.CompilerParams(use_tc_tiling_on_sc=False),
      scratch_types=[
          pltpu.VMEM((N_CHUNKS, CHUNK_IDX), jnp.int32),
          pltpu.VMEM((CHUNK_IDX, EMBED_DIM), jnp.float32),
          pltpu.VMEM((CHUNK_IDX, EMBED_DIM), jnp.float32),
          pltpu.VMEM((B_PER_W, EMBED_DIM), jnp.float32),
          pltpu.SemaphoreType.DMA,
          pltpu.SemaphoreType.DMA,
      ],
  )(_sc_body)
  out = run(x3, table)
  return out.reshape(BATCH, EMBED_DIM)
